# baseline (speedup 1.0000x reference)
import functools

import jax
import jax.numpy as jnp
from jax.experimental import pallas as pl
from jax.experimental.pallas import tpu as pltpu

NUM_EXPERTS = 64
NUM_HASH = 4
HIDDEN = 768
HIDDEN_Q = HIDDEN // 4
_CHUNK_M = 1024


def _routing_kernel(x_ref, w1t_ref, pack_ref,
                    rw_ref, ea_ref, hash_ref, lb_ref,
                    w1s_ref, w2s_ref, b2s_ref):
    @pl.when(pl.program_id(0) == 0)
    def _prep():
        for k in range(NUM_HASH):
            w1s_ref[:, k * HIDDEN_Q:(k + 1) * HIDDEN_Q] = (
                w1t_ref[k].T.astype(jnp.bfloat16))
        w2col = jnp.transpose(pack_ref[0:1, :])
        rows = jax.lax.broadcasted_iota(jnp.int32, (HIDDEN, NUM_HASH), 0)
        cols = jax.lax.broadcasted_iota(jnp.int32, (HIDDEN, NUM_HASH), 1)
        w2s_ref[...] = jnp.where(rows // HIDDEN_Q == cols, w2col,
                                 0.0).astype(jnp.bfloat16)
        b2s_ref[...] = jnp.transpose(pack_ref[2:3, :NUM_HASH])
        lb_ref[...] = jnp.zeros_like(lb_ref)

    block_t = x_ref.shape[0]
    lb_acc = jnp.zeros((NUM_EXPERTS,), jnp.float32)
    for m in range(0, block_t, _CHUNK_M):
        x = x_ref[m:m + _CHUNK_M, :].astype(jnp.bfloat16)
        h = jnp.dot(x, w1s_ref[...], preferred_element_type=jnp.float32)
        h = jnp.maximum(h + pack_ref[1:2, :], 0.0)
        hv = jnp.dot(h.astype(jnp.bfloat16), w2s_ref[...],
                     preferred_element_type=jnp.float32)
        hvT = jnp.transpose(hv) + b2s_ref[...]
        hash_ref[:, m:m + _CHUNK_M] = hvT
        summed = jnp.sum(hvT, axis=0, keepdims=True)
        e = summed.astype(jnp.int32)
        r = jnp.bitwise_and(e, NUM_EXPERTS - 1)
        ea_ref[m:m + _CHUNK_M] = r.reshape(_CHUNK_M)
        subl = jax.lax.broadcasted_iota(
            jnp.int32, (NUM_EXPERTS, _CHUNK_M), 0)
        rwT = (subl == r).astype(jnp.float32)
        rw_ref[:, m:m + _CHUNK_M] = rwT
        lb_acc = lb_acc + jnp.sum(rwT, axis=1)
    lb_ref[...] += lb_acc


@functools.partial(jax.jit, static_argnames=("block_t",))
def _run(hidden_flat, W1t, pack, block_t):
    T = hidden_flat.shape[0]
    grid = (T // block_t,)
    rwT, ea, hashesT, lb = pl.pallas_call(
        _routing_kernel,
        grid=grid,
        in_specs=[
            pl.BlockSpec((block_t, HIDDEN), lambda i: (i, 0)),
            pl.BlockSpec((NUM_HASH, HIDDEN_Q, HIDDEN), lambda i: (0, 0, 0)),
            pl.BlockSpec((3, HIDDEN), lambda i: (0, 0)),
        ],
        out_specs=[
            pl.BlockSpec((NUM_EXPERTS, block_t), lambda i: (0, i)),
            pl.BlockSpec((block_t,), lambda i: (i,)),
            pl.BlockSpec((NUM_HASH, block_t), lambda i: (0, i)),
            pl.BlockSpec((NUM_EXPERTS,), lambda i: (0,)),
        ],
        out_shape=[
            jax.ShapeDtypeStruct((NUM_EXPERTS, T), jnp.float32),
            jax.ShapeDtypeStruct((T,), jnp.int32),
            jax.ShapeDtypeStruct((NUM_HASH, T), jnp.float32),
            jax.ShapeDtypeStruct((NUM_EXPERTS,), jnp.float32),
        ],
        scratch_shapes=[
            pltpu.VMEM((HIDDEN, NUM_HASH * HIDDEN_Q), jnp.bfloat16),
            pltpu.VMEM((NUM_HASH * HIDDEN_Q, NUM_HASH), jnp.bfloat16),
            pltpu.VMEM((NUM_HASH, 1), jnp.float32),
        ],
    )(hidden_flat, W1t, pack)
    return rwT, ea, hashesT, lb


def kernel(hidden_states, W1, b1, W2, b2):
    B, S, H = hidden_states.shape
    hidden_flat = hidden_states.reshape(-1, H)
    W1t = jnp.transpose(W1, (0, 2, 1))
    pack = jnp.concatenate([
        W2.reshape(1, H),
        b1.reshape(1, H),
        jnp.pad(b2.reshape(1, NUM_HASH), ((0, 0), (0, H - NUM_HASH))),
    ], axis=0)
    rwT, ea, hashesT, lb = _run(hidden_flat, W1t, pack, block_t=2048)
    return rwT.T, ea[:, None], hashesT.T, lb


# speedup vs baseline: 1.0220x; 1.0220x over previous
import functools

import jax
import jax.numpy as jnp
from jax.experimental import pallas as pl
from jax.experimental.pallas import tpu as pltpu

NUM_EXPERTS = 64
NUM_HASH = 4
HIDDEN = 768
HIDDEN_Q = HIDDEN // 4
_CHUNK_M = 1024


def _routing_kernel(x_ref, w1t_ref, pack_ref,
                    rw_ref, ea_ref, hash_ref, lb_ref,
                    w1s_ref, w2s_ref, b2s_ref):
    @pl.when(pl.program_id(0) == 0)
    def _prep():
        for k in range(NUM_HASH):
            w1s_ref[:, k * HIDDEN_Q:(k + 1) * HIDDEN_Q] = (
                w1t_ref[k].T.astype(jnp.bfloat16))
        w2col = jnp.transpose(pack_ref[0:1, :])
        rows = jax.lax.broadcasted_iota(jnp.int32, (HIDDEN, NUM_HASH), 0)
        cols = jax.lax.broadcasted_iota(jnp.int32, (HIDDEN, NUM_HASH), 1)
        w2s_ref[...] = jnp.where(rows // HIDDEN_Q == cols, w2col,
                                 0.0).astype(jnp.bfloat16)
        b2s_ref[...] = jnp.transpose(pack_ref[2:3, :NUM_HASH])
        lb_ref[...] = jnp.zeros_like(lb_ref)

    block_t = x_ref.shape[0]
    lb_acc = jnp.zeros((NUM_EXPERTS,), jnp.float32)
    for m in range(0, block_t, _CHUNK_M):
        x = x_ref[m:m + _CHUNK_M, :].astype(jnp.bfloat16)
        h = jnp.dot(x, w1s_ref[...], preferred_element_type=jnp.float32)
        h = jnp.maximum(h + pack_ref[1:2, :], 0.0)
        hv = jnp.dot(h.astype(jnp.bfloat16), w2s_ref[...],
                     preferred_element_type=jnp.float32)
        hvT = jnp.transpose(hv) + b2s_ref[...]
        hash_ref[:, m:m + _CHUNK_M] = hvT
        summed = jnp.sum(hvT, axis=0, keepdims=True)
        e = summed.astype(jnp.int32)
        r = jnp.bitwise_and(e, NUM_EXPERTS - 1)
        ea_ref[m:m + _CHUNK_M] = r.reshape(_CHUNK_M)
        subl = jax.lax.broadcasted_iota(
            jnp.int32, (NUM_EXPERTS, _CHUNK_M), 0)
        rwT = (subl == r).astype(jnp.float32)
        rw_ref[:, m:m + _CHUNK_M] = rwT
        lb_acc = lb_acc + jnp.sum(rwT, axis=1)
    lb_ref[...] += lb_acc


@functools.partial(jax.jit, static_argnames=("block_t",))
def _run(hidden_flat, W1t, pack, block_t):
    T = hidden_flat.shape[0]
    grid = (T // block_t,)
    rwT, ea, hashesT, lb = pl.pallas_call(
        _routing_kernel,
        grid=grid,
        in_specs=[
            pl.BlockSpec((block_t, HIDDEN), lambda i: (i, 0)),
            pl.BlockSpec((NUM_HASH, HIDDEN_Q, HIDDEN), lambda i: (0, 0, 0)),
            pl.BlockSpec((3, HIDDEN), lambda i: (0, 0)),
        ],
        out_specs=[
            pl.BlockSpec((NUM_EXPERTS, block_t), lambda i: (0, i)),
            pl.BlockSpec((block_t,), lambda i: (i,)),
            pl.BlockSpec((NUM_HASH, block_t), lambda i: (0, i)),
            pl.BlockSpec((NUM_EXPERTS,), lambda i: (0,)),
        ],
        out_shape=[
            jax.ShapeDtypeStruct((NUM_EXPERTS, T), jnp.float32),
            jax.ShapeDtypeStruct((T,), jnp.int32),
            jax.ShapeDtypeStruct((NUM_HASH, T), jnp.float32),
            jax.ShapeDtypeStruct((NUM_EXPERTS,), jnp.float32),
        ],
        compiler_params=pltpu.CompilerParams(
            vmem_limit_bytes=117 * 1024 * 1024),
        scratch_shapes=[
            pltpu.VMEM((HIDDEN, NUM_HASH * HIDDEN_Q), jnp.bfloat16),
            pltpu.VMEM((NUM_HASH * HIDDEN_Q, NUM_HASH), jnp.bfloat16),
            pltpu.VMEM((NUM_HASH, 1), jnp.float32),
        ],
    )(hidden_flat, W1t, pack)
    return rwT, ea, hashesT, lb


def kernel(hidden_states, W1, b1, W2, b2):
    B, S, H = hidden_states.shape
    hidden_flat = hidden_states.reshape(-1, H)
    W1t = jnp.transpose(W1, (0, 2, 1))
    pack = jnp.concatenate([
        W2.reshape(1, H),
        b1.reshape(1, H),
        jnp.pad(b2.reshape(1, NUM_HASH), ((0, 0), (0, H - NUM_HASH))),
    ], axis=0)
    rwT, ea, hashesT, lb = _run(hidden_flat, W1t, pack, block_t=8192)
    return rwT.T, ea[:, None], hashesT.T, lb


# speedup vs baseline: 1.0444x; 1.0219x over previous
import functools

import jax
import jax.numpy as jnp
from jax.experimental import pallas as pl
from jax.experimental.pallas import tpu as pltpu

NUM_EXPERTS = 64
NUM_HASH = 4
HIDDEN = 768
HIDDEN_Q = HIDDEN // 4
_CHUNK_M = 1024


def _routing_kernel(x_ref, w1t_ref, pack_ref,
                    rw_ref, ea_ref, hash_ref, lb_ref,
                    w1s_ref, w2s_ref, b2s_ref):
    @pl.when(pl.program_id(0) == 0)
    def _prep():
        for k in range(NUM_HASH):
            w1s_ref[:, k * HIDDEN_Q:(k + 1) * HIDDEN_Q] = (
                w1t_ref[k].T.astype(jnp.bfloat16))
        w2col = jnp.transpose(pack_ref[0:1, :])
        rows = jax.lax.broadcasted_iota(jnp.int32, (HIDDEN, NUM_HASH), 0)
        cols = jax.lax.broadcasted_iota(jnp.int32, (HIDDEN, NUM_HASH), 1)
        w2s_ref[...] = jnp.where(rows // HIDDEN_Q == cols, w2col,
                                 0.0).astype(jnp.bfloat16)
        b2s_ref[...] = jnp.transpose(pack_ref[2:3, :NUM_HASH])
        lb_ref[...] = jnp.zeros_like(lb_ref)

    block_t = x_ref.shape[0]
    lb_acc = jnp.zeros((NUM_EXPERTS,), jnp.float32)
    for m in range(0, block_t, _CHUNK_M):
        x = x_ref[m:m + _CHUNK_M, :].astype(jnp.bfloat16)
        h = jnp.dot(x, w1s_ref[...], preferred_element_type=jnp.float32)
        h = jnp.maximum(h + pack_ref[1:2, :], 0.0)
        hv = jnp.dot(h.astype(jnp.bfloat16), w2s_ref[...],
                     preferred_element_type=jnp.float32)
        hvT = jnp.transpose(hv) + b2s_ref[...]
        hash_ref[:, m:m + _CHUNK_M] = hvT
        summed = jnp.sum(hvT, axis=0, keepdims=True)
        e = summed.astype(jnp.int32)
        r = jnp.bitwise_and(e, NUM_EXPERTS - 1)
        ea_ref[m:m + _CHUNK_M] = r.reshape(_CHUNK_M)
        subl = jax.lax.broadcasted_iota(
            jnp.int32, (NUM_EXPERTS, _CHUNK_M), 0)
        rwT = (subl == r).astype(jnp.float32)
        rw_ref[:, m:m + _CHUNK_M] = rwT
        lb_acc = lb_acc + jnp.sum(rwT, axis=1)
    lb_ref[...] += lb_acc


@functools.partial(jax.jit, static_argnames=("block_t",))
def _run(hidden_flat, W1t, pack, block_t):
    T = hidden_flat.shape[0]
    grid = (T // block_t,)
    rwT, ea, hashesT, lb = pl.pallas_call(
        _routing_kernel,
        grid=grid,
        in_specs=[
            pl.BlockSpec((block_t, HIDDEN), lambda i: (i, 0)),
            pl.BlockSpec((NUM_HASH, HIDDEN_Q, HIDDEN), lambda i: (0, 0, 0)),
            pl.BlockSpec((3, HIDDEN), lambda i: (0, 0)),
        ],
        out_specs=[
            pl.BlockSpec((NUM_EXPERTS, block_t), lambda i: (0, i)),
            pl.BlockSpec((block_t,), lambda i: (i,)),
            pl.BlockSpec((NUM_HASH, block_t), lambda i: (0, i)),
            pl.BlockSpec((NUM_EXPERTS,), lambda i: (0,)),
        ],
        out_shape=[
            jax.ShapeDtypeStruct((NUM_EXPERTS, T), jnp.float32),
            jax.ShapeDtypeStruct((T,), jnp.int32),
            jax.ShapeDtypeStruct((NUM_HASH, T), jnp.float32),
            jax.ShapeDtypeStruct((NUM_EXPERTS,), jnp.float32),
        ],
        scratch_shapes=[
            pltpu.VMEM((HIDDEN, NUM_HASH * HIDDEN_Q), jnp.bfloat16),
            pltpu.VMEM((NUM_HASH * HIDDEN_Q, NUM_HASH), jnp.bfloat16),
            pltpu.VMEM((NUM_HASH, 1), jnp.float32),
        ],
    )(hidden_flat, W1t, pack)
    return rwT, ea, hashesT, lb


def kernel(hidden_states, W1, b1, W2, b2):
    B, S, H = hidden_states.shape
    hidden_flat = hidden_states.reshape(-1, H)
    W1t = jnp.transpose(W1, (0, 2, 1))
    pack = jnp.concatenate([
        W2.reshape(1, H),
        b1.reshape(1, H),
        jnp.pad(b2.reshape(1, NUM_HASH), ((0, 0), (0, H - NUM_HASH))),
    ], axis=0)
    rwT, ea, hashesT, lb = _run(hidden_flat, W1t, pack, block_t=4096)
    return rwT.T, ea[:, None], hashesT.T, lb
